# baseline (speedup 1.0000x reference)
import functools

import jax
import jax.numpy as jnp
from jax import lax
from jax.experimental import pallas as pl
from jax.experimental.pallas import tpu as pltpu
from jax.experimental.pallas import tpu_sc as plsc

_DIM = 64
_B = 4096
_NC, _NS, _L = 2, 16, 16
_NW = _NC * _NS
_BPW = _B // _NW
_NU = 100000


@functools.cache
def _make_gather_sc():
  mesh = plsc.VectorSubcoreMesh(
      core_axis_name="c", subcore_axis_name="s",
      num_cores=_NC, num_subcores=_NS)

  @functools.partial(
      pl.kernel,
      mesh=mesh,
      out_type=(
          jax.ShapeDtypeStruct((_B, _DIM), jnp.float32),
          jax.ShapeDtypeStruct((_B, _DIM), jnp.float32),
          jax.ShapeDtypeStruct((_B, _DIM), jnp.float32),
          jax.ShapeDtypeStruct((_B, _DIM), jnp.float32),
          jax.ShapeDtypeStruct((_B,), jnp.float32),
          jax.ShapeDtypeStruct((_B,), jnp.float32),
      ),
      scratch_types=[
          pltpu.VMEM((_BPW,), jnp.int32),
          pltpu.VMEM((_BPW,), jnp.int32),
          pltpu.VMEM((_BPW,), jnp.int32),
          pltpu.VMEM((_BPW, _DIM), jnp.float32),
          pltpu.VMEM((_BPW, _DIM), jnp.float32),
          pltpu.VMEM((_BPW, _DIM), jnp.float32),
          pltpu.VMEM((_BPW, _DIM), jnp.float32),
          pltpu.VMEM((_BPW,), jnp.float32),
          pltpu.VMEM((_BPW,), jnp.float32),
          pltpu.SemaphoreType.DMA,
      ],
  )
  def _gather_sc(uid_hbm, iid_hbm, iidw_hbm, user_emb, item_emb, word_emb,
                 bias_u1, bias_i1,
                 ue_o, ug_o, ie_o, ig_o, bu_o, bi_o,
                 uid_v, iid_v, iidw_v, ue_v, ug_v, ie_v, ig_v, bu_v, bi_v, sem):
    wid = lax.axis_index("s") * _NC + lax.axis_index("c")
    base = wid * _BPW
    pltpu.sync_copy(uid_hbm.at[pl.ds(base, _BPW)], uid_v)
    pltpu.sync_copy(iid_hbm.at[pl.ds(base, _BPW)], iid_v)
    pltpu.sync_copy(iidw_hbm.at[pl.ds(base, _BPW)], iidw_v)

    def body(g, _):
      off = g * _L
      uvec = uid_v[pl.ds(off, _L)]
      ivec = iid_v[pl.ds(off, _L)]
      wvec = iidw_v[pl.ds(off, _L)]
      descs = []
      for l in range(_L):
        r = off + l
        descs.append(pltpu.async_copy(user_emb.at[uvec[l]], ue_v.at[r], sem))
        descs.append(pltpu.async_copy(word_emb.at[uvec[l]], ug_v.at[r], sem))
        descs.append(pltpu.async_copy(item_emb.at[ivec[l]], ie_v.at[r], sem))
        descs.append(pltpu.async_copy(word_emb.at[wvec[l]], ig_v.at[r], sem))
      for d in descs:
        d.wait()
      return 0

    lax.fori_loop(0, _BPW // _L, body, 0)
    pltpu.async_copy(bias_u1.at[uid_v], bu_v, sem).wait()
    pltpu.async_copy(bias_i1.at[iid_v], bi_v, sem).wait()
    pltpu.sync_copy(ue_v, ue_o.at[pl.ds(base, _BPW)])
    pltpu.sync_copy(ug_v, ug_o.at[pl.ds(base, _BPW)])
    pltpu.sync_copy(ie_v, ie_o.at[pl.ds(base, _BPW)])
    pltpu.sync_copy(ig_v, ig_o.at[pl.ds(base, _BPW)])
    pltpu.sync_copy(bu_v, bu_o.at[pl.ds(base, _BPW)])
    pltpu.sync_copy(bi_v, bi_o.at[pl.ds(base, _BPW)])

  return _gather_sc


def _fm_tc(ue, ug, ie, ig, v_ref, wrow_ref, bu, bi, bias_ref, out_ref):
  x = jnp.concatenate([ue[...], ug[...], ie[...], ig[...]], axis=1)
  v = v_ref[...]
  y = jnp.dot(x, v, preferred_element_type=jnp.float32)
  sv = jnp.sum(v * v, axis=1)
  lin = jnp.sum(x * wrow_ref[...], axis=1)
  t1 = jnp.sum(y * y, axis=1)
  t2 = jnp.sum((x * x) * sv[None, :], axis=1)
  out_ref[...] = lin + 0.5 * (t1 - t2) + bu[...] + bi[...] + bias_ref[0]


_fm_call = pl.pallas_call(
    _fm_tc,
    in_specs=[
        pl.BlockSpec(memory_space=pltpu.VMEM),
        pl.BlockSpec(memory_space=pltpu.VMEM),
        pl.BlockSpec(memory_space=pltpu.VMEM),
        pl.BlockSpec(memory_space=pltpu.VMEM),
        pl.BlockSpec(memory_space=pltpu.VMEM),
        pl.BlockSpec(memory_space=pltpu.VMEM),
        pl.BlockSpec(memory_space=pltpu.VMEM),
        pl.BlockSpec(memory_space=pltpu.VMEM),
        pl.BlockSpec(memory_space=pltpu.SMEM),
    ],
    out_shape=jax.ShapeDtypeStruct((_B,), jnp.float32),
)


def kernel(uid_batch, iid_batch, u_nodes, u_adj_ind, u_adj_tp,
           i_nodes, i_adj_ind, i_adj_tp,
           user_emb, item_emb, word_emb, W_lin, V, bias_u, bias_i, bias):
  uid = uid_batch.astype(jnp.int32)
  iid = iid_batch.astype(jnp.int32)
  ue, ug, ie, ig, bu, bi = _make_gather_sc()(
      uid, iid, iid + _NU, user_emb, item_emb, word_emb, bias_u, bias_i)
  return _fm_call(ue, ug, ie, ig, V, W_lin.reshape(1, -1), bu, bi, bias)
